# baseline (speedup 1.0000x reference)
import functools

import jax
import jax.numpy as jnp
from jax import lax
from jax.experimental import pallas as pl
from jax.experimental.pallas import tpu as pltpu
from jax.experimental.pallas import tpu_sc as plsc

_VOCAB = 100000
_EMBED = 128
_BATCH = 1024
_CTX = 50

_NC = 2
_NS = 16
_LANES = 16
_NW = _NC * _NS
_B_PER_W = _BATCH // _NW
_EV = _EMBED // _LANES


_RB = 6


def _sc_pool_body(emb_hbm, idx_hbm, out_hbm, idx_v, rows_v, pool_v, sems):
    wid = lax.axis_index("s") * _NC + lax.axis_index("c")
    base = wid * _B_PER_W
    pltpu.sync_copy(idx_hbm.at[pl.ds(base, _B_PER_W)], idx_v)

    def gather(b, k):
        return pltpu.make_async_copy(
            emb_hbm.at[idx_v.at[b]], rows_v.at[k], sems.at[k]
        )

    for k in range(_RB - 1):
        gather(k, k).start()

    def do_row(b, carry):
        k = lax.rem(b, _RB)
        gather(b, k).wait()
        nb = b + _RB - 1

        @pl.when(nb < _B_PER_W)
        def _prefetch():
            gather(nb, lax.rem(nb, _RB)).start()

        scale = 1.0 / _CTX
        for j in range(_EV):
            acc = rows_v[k, 0, pl.ds(j * _LANES, _LANES)]
            for c in range(1, _CTX):
                acc = acc + rows_v[k, c, pl.ds(j * _LANES, _LANES)]
            pool_v[b, pl.ds(j * _LANES, _LANES)] = acc * scale
        return carry

    lax.fori_loop(0, _B_PER_W, do_row, 0)
    pltpu.sync_copy(pool_v, out_hbm.at[pl.ds(base, _B_PER_W)])


@jax.jit
def _sc_pool(emb_table, idx):
    mesh = plsc.VectorSubcoreMesh(core_axis_name="c", subcore_axis_name="s")
    return pl.kernel(
        _sc_pool_body,
        out_type=jax.ShapeDtypeStruct((_BATCH, _EMBED), jnp.float32),
        mesh=mesh,
        scratch_types=[
            pltpu.VMEM((_B_PER_W, _CTX), jnp.int32),
            pltpu.VMEM((_RB, _CTX, _EMBED), jnp.float32),
            pltpu.VMEM((_B_PER_W, _EMBED), jnp.float32),
            pltpu.SemaphoreType.DMA((_RB,)),
        ],
    )(emb_table, idx)


_VT = 5000


def _matmul_body(wt_ref, x_ref, b_ref, o_ref):
    acc = jax.lax.dot_general(
        wt_ref[...],
        x_ref[...],
        (((1,), (1,)), ((), ())),
        preferred_element_type=jnp.float32,
    )
    o_ref[...] = acc + jnp.reshape(b_ref[...], (_VT, 1))


@jax.jit
def _project(pooled, dense_w, dense_b):
    wt = dense_w.T
    out_t = pl.pallas_call(
        _matmul_body,
        grid=(_VOCAB // _VT,),
        in_specs=[
            pl.BlockSpec((_VT, _EMBED), lambda i: (i, 0)),
            pl.BlockSpec((_BATCH, _EMBED), lambda i: (0, 0)),
            pl.BlockSpec((1, 1, _VT), lambda i: (i, 0, 0)),
        ],
        out_specs=pl.BlockSpec((_VT, _BATCH), lambda i: (i, 0)),
        out_shape=jax.ShapeDtypeStruct((_VOCAB, _BATCH), jnp.float32),
    )(wt, pooled, dense_b.reshape(_VOCAB // _VT, 1, _VT))
    return out_t.T


def kernel(inputs, emb_table, dense_w, dense_b):
    idx = inputs.astype(jnp.int32)
    pooled = _sc_pool(emb_table, idx)
    return _project(pooled, dense_w, dense_b)


# speedup vs baseline: 1.0035x; 1.0035x over previous
import jax
import jax.numpy as jnp
from jax import lax
from jax.experimental import pallas as pl
from jax.experimental.pallas import tpu as pltpu
from jax.experimental.pallas import tpu_sc as plsc

_VOCAB = 100000
_EMBED = 128
_BATCH = 1024
_CTX = 50

_NC = 2
_NS = 16
_LANES = 16
_NW = _NC * _NS
_B_PER_W = _BATCH // _NW
_EV = _EMBED // _LANES


_RB = 4


def _sc_pool_body(emb_hbm, idx_hbm, out_hbm, idx_v, rows_v, pool_v, sems):
    wid = lax.axis_index("s") * _NC + lax.axis_index("c")
    base = wid * _B_PER_W
    pltpu.sync_copy(idx_hbm.at[pl.ds(base, _B_PER_W)], idx_v)

    def gather(b, k):
        return pltpu.make_async_copy(
            emb_hbm.at[idx_v.at[b]], rows_v.at[k], sems.at[k]
        )

    for k in range(_RB - 1):
        gather(k, k).start()

    def do_row(b, carry):
        k = lax.rem(b, _RB)
        gather(b, k).wait()
        nb = b + _RB - 1

        @pl.when(nb < _B_PER_W)
        def _prefetch():
            gather(nb, lax.rem(nb, _RB)).start()

        scale = 1.0 / _CTX
        for j in range(_EV):
            acc = rows_v[k, 0, pl.ds(j * _LANES, _LANES)]
            for c in range(1, _CTX):
                acc = acc + rows_v[k, c, pl.ds(j * _LANES, _LANES)]
            pool_v[b, pl.ds(j * _LANES, _LANES)] = acc * scale
        return carry

    lax.fori_loop(0, _B_PER_W, do_row, 0)
    pltpu.sync_copy(pool_v, out_hbm.at[pl.ds(base, _B_PER_W)])


@jax.jit
def _sc_pool(emb_table, idx):
    mesh = plsc.VectorSubcoreMesh(core_axis_name="c", subcore_axis_name="s")
    return pl.kernel(
        _sc_pool_body,
        out_type=jax.ShapeDtypeStruct((_BATCH, _EMBED), jnp.float32),
        mesh=mesh,
        scratch_types=[
            pltpu.VMEM((_B_PER_W, _CTX), jnp.int32),
            pltpu.VMEM((_RB, _CTX, _EMBED), jnp.float32),
            pltpu.VMEM((_B_PER_W, _EMBED), jnp.float32),
            pltpu.SemaphoreType.DMA((_RB,)),
        ],
    )(emb_table, idx)


_VT = 5000


def _matmul_body(wt_ref, x_ref, b_ref, o_ref):
    acc = jax.lax.dot_general(
        wt_ref[...],
        x_ref[...],
        (((1,), (1,)), ((), ())),
        preferred_element_type=jnp.float32,
    )
    o_ref[...] = acc + jnp.reshape(b_ref[...], (_VT, 1))


@jax.jit
def _project(pooled, dense_w, dense_b):
    wt = dense_w.T
    out_t = pl.pallas_call(
        _matmul_body,
        grid=(_VOCAB // _VT,),
        in_specs=[
            pl.BlockSpec((_VT, _EMBED), lambda i: (i, 0)),
            pl.BlockSpec((_BATCH, _EMBED), lambda i: (0, 0)),
            pl.BlockSpec((1, 1, _VT), lambda i: (i, 0, 0)),
        ],
        out_specs=pl.BlockSpec((_VT, _BATCH), lambda i: (i, 0)),
        out_shape=jax.ShapeDtypeStruct((_VOCAB, _BATCH), jnp.float32),
    )(wt, pooled, dense_b.reshape(_VOCAB // _VT, 1, _VT))
    return out_t.T


def kernel(inputs, emb_table, dense_w, dense_b):
    idx = inputs.astype(jnp.int32)
    pooled = _sc_pool(emb_table, idx)
    return _project(pooled, dense_w, dense_b)
